# baseline (speedup 1.0000x reference)
import functools

import jax
import jax.numpy as jnp
from jax import lax
from jax.experimental import pallas as pl
from jax.experimental.pallas import tpu as pltpu
from jax.experimental.pallas import tpu_sc as plsc

DIMS = (64, 64, 64)
FEAT = 64
BATCH = 16384
NROWS = DIMS[0] * DIMS[1] * DIMS[2]

NUM_CORES = 2
NUM_SUBCORES = 16
LANES = 16
NUM_WORKERS = NUM_CORES * NUM_SUBCORES
BPW = BATCH // NUM_WORKERS
KBATCH = 128
NBATCH = BPW // KBATCH
ROW_BYTES = FEAT * 4

_mesh = plsc.VectorSubcoreMesh(core_axis_name="c", subcore_axis_name="s")


@functools.partial(
    pl.kernel,
    mesh=_mesh,
    out_type=jax.ShapeDtypeStruct((BATCH, FEAT), jnp.float32),
    scratch_types=[
        pltpu.VMEM((3 * BPW,), jnp.int32),
        pltpu.VMEM((BPW,), jnp.int32),
        pltpu.VMEM((BPW, FEAT), jnp.float32),
        pltpu.SemaphoreType.DMA,
        pltpu.SemaphoreType.DMA,
    ],
)
def _lookup(idx_hbm, tab_hbm, out_hbm, raw_v, flat_v, rows_v, sem, wsem):
    wid = lax.axis_index("s") * NUM_CORES + lax.axis_index("c")
    base = wid * BPW

    stage = [
        pltpu.async_copy(idx_hbm.at[pl.ds(c * BATCH + base, BPW)],
                         raw_v.at[pl.ds(c * BPW, BPW)], sem)
        for c in range(3)
    ]
    for cp in stage:
        cp.wait()

    for g in range(BPW // LANES):
        o16 = g * LANES
        i0 = raw_v[pl.ds(o16, LANES)]
        i1 = raw_v[pl.ds(BPW + o16, LANES)]
        i2 = raw_v[pl.ds(2 * BPW + o16, LANES)]
        flat_v[pl.ds(o16, LANES)] = (
            i0 * (DIMS[1] * DIMS[2]) + i1 * DIMS[2] + i2
        )

    def fire(g):
        r0 = g * KBATCH
        copies = []
        for h in range(KBATCH // LANES):
            fv = flat_v[pl.ds(r0 + h * LANES, LANES)]
            for l in range(LANES):
                r = r0 + h * LANES + l
                copies.append(
                    pltpu.async_copy(tab_hbm.at[fv[l]], rows_v.at[r], sem)
                )
        return copies

    inflight = {g: fire(g) for g in range(min(2, NBATCH))}
    writes = []
    for g in range(NBATCH):
        if g + 2 < NBATCH:
            inflight[g + 2] = fire(g + 2)
        for cp in inflight.pop(g):
            cp.wait()
        writes.append(
            pltpu.async_copy(
                rows_v.at[pl.ds(g * KBATCH, KBATCH)],
                out_hbm.at[pl.ds(base + g * KBATCH, KBATCH)],
                wsem,
            )
        )
    for cp in writes:
        cp.wait()


def kernel(indices, table):
    idx_cols = indices.astype(jnp.int32).T.reshape(-1)
    tab2d = table.reshape(NROWS, FEAT)
    return _lookup(idx_cols, tab2d)


# speedup vs baseline: 1.1345x; 1.1345x over previous
import functools

import jax
import jax.numpy as jnp
from jax import lax
from jax.experimental import pallas as pl
from jax.experimental.pallas import tpu as pltpu
from jax.experimental.pallas import tpu_sc as plsc

DIMS = (64, 64, 64)
FEAT = 64
BATCH = 16384
NROWS = DIMS[0] * DIMS[1] * DIMS[2]

NUM_CORES = 2
NUM_SUBCORES = 16
LANES = 16
NUM_WORKERS = NUM_CORES * NUM_SUBCORES
BPW = BATCH // NUM_WORKERS
KBATCH = 128
NBATCH = BPW // KBATCH
ROW_BYTES = FEAT * 4

_mesh = plsc.VectorSubcoreMesh(core_axis_name="c", subcore_axis_name="s")


@functools.partial(
    pl.kernel,
    mesh=_mesh,
    out_type=jax.ShapeDtypeStruct((BATCH, FEAT), jnp.float32),
    scratch_types=[
        pltpu.VMEM((3 * BPW,), jnp.int32),
        pltpu.VMEM((BPW,), jnp.int32),
        pltpu.VMEM((BPW, FEAT), jnp.float32),
        pltpu.SemaphoreType.DMA,
        pltpu.SemaphoreType.DMA,
    ],
)
def _lookup(idx_hbm, tab_hbm, out_hbm, raw_v, flat_v, rows_v, sem, wsem):
    wid = lax.axis_index("s") * NUM_CORES + lax.axis_index("c")
    base = wid * BPW

    stage = [
        pltpu.async_copy(idx_hbm.at[pl.ds(c * BATCH + base, BPW)],
                         raw_v.at[pl.ds(c * BPW, BPW)], sem)
        for c in range(3)
    ]
    for cp in stage:
        cp.wait()

    for g in range(BPW // LANES):
        o16 = g * LANES
        i0 = raw_v[pl.ds(o16, LANES)]
        i1 = raw_v[pl.ds(BPW + o16, LANES)]
        i2 = raw_v[pl.ds(2 * BPW + o16, LANES)]
        flat_v[pl.ds(o16, LANES)] = (
            i0 * (DIMS[1] * DIMS[2]) + i1 * DIMS[2] + i2
        )

    @pl.loop(0, NBATCH)
    def _batch(g):
        r0 = g * KBATCH
        copies = []
        for h in range(KBATCH // LANES):
            fv = flat_v[pl.ds(r0 + h * LANES, LANES)]
            for l in range(LANES):
                r = r0 + h * LANES + l
                copies.append(
                    pltpu.async_copy(tab_hbm.at[fv[l]], rows_v.at[r], sem)
                )
        for cp in copies:
            cp.wait()

    pltpu.sync_copy(rows_v, out_hbm.at[pl.ds(base, BPW)])


def kernel(indices, table):
    idx_cols = indices.astype(jnp.int32).T.reshape(-1)
    tab2d = table.reshape(NROWS, FEAT)
    return _lookup(idx_cols, tab2d)
